# baseline (speedup 1.0000x reference)
import functools

import jax
import jax.numpy as jnp
from jax import lax
from jax.experimental import pallas as pl
from jax.experimental.pallas import tpu as pltpu
from jax.experimental.pallas import tpu_sc as plsc

NC = 2
NS = 16
NW = NC * NS
L = 16

B = 1024
W = 25
D = 25
WD = W * D
E = 128
V = 16
NE = 9
NI = 6
GR = 3 * V * V + 3 * V
NG = 40
PWD = NG * L
ECE = 8
NEC = E // ECE


def _body(gt_hbm, idx_hbm, out_hbm, gt_v, idx_v, acc_v, sem0, sem1):
    wid = lax.axis_index("s") * NC + lax.axis_index("c")
    pltpu.sync_copy(gt_hbm, gt_v)
    sems = [sem0, sem1]

    def task_body(t, carry):
        b = t * NW + wid
        pltpu.sync_copy(idx_hbm.at[b], idx_v)

        def chunk2_body(c2, cc):
            for half in range(2):
                ec = c2 * 2 + half
                sem = sems[half]
                dst = out_hbm.at[b, pl.ds(ec * ECE, ECE)]

                @pl.when((c2 > 0) | (t > 0))
                def _wait_prev(half=half, dst=dst, sem=sem):
                    pltpu.make_async_copy(acc_v.at[half], dst, sem).wait()

                @plsc.parallel_loop(0, NG, unroll=8)
                def g_body(g, ec=ec, half=half):
                    o = lax.min(g * L, WD - L)
                    ivs = [idx_v[i, pl.ds(o, L)] for i in range(NI)]
                    for j in range(ECE):
                        base = (ec * ECE + j) * GR
                        win = gt_v.at[pl.ds(base, GR)]
                        gv = [plsc.load_gather(win, [ivs[i]])
                              for i in range(NI)]
                        s = (((gv[0] + gv[1]) + (gv[2] + gv[3]))
                             + (gv[4] + gv[5]))
                        acc_v[half, j, pl.ds(o, L)] = s

                pltpu.async_copy(acc_v.at[half], dst, sem)
            return cc

        lax.fori_loop(0, NEC // 2, chunk2_body, 0)
        return carry

    lax.fori_loop(0, B // NW, task_body, 0)

    for bufi in range(2):
        pltpu.make_async_copy(
            acc_v.at[bufi], out_hbm.at[0, pl.ds(bufi * ECE, ECE)],
            sems[bufi]).wait()


@jax.jit
def _embed(gtT, idx6):
    mesh = plsc.VectorSubcoreMesh(core_axis_name="c", subcore_axis_name="s")
    f = pl.kernel(
        _body,
        out_type=jax.ShapeDtypeStruct((B, E, WD), jnp.float32),
        mesh=mesh,
        scratch_types=[
            pltpu.VMEM((E * GR,), jnp.float32),
            pltpu.VMEM((NI, PWD), jnp.int32),
            pltpu.VMEM((2, ECE, WD), jnp.float32),
            pltpu.SemaphoreType.DMA,
            pltpu.SemaphoreType.DMA,
        ],
        compiler_params=pltpu.CompilerParams(needs_layout_passes=False),
    )
    return f(gtT, idx6)


def kernel(prop_types, hut_colors, hut_rotations, tree_types, plant_types,
           windmill_rotations, tower_rotations, tent_rotations, terrain,
           table_0, table_1, table_2, table_3, table_4, table_5, table_6,
           table_7, table_8):
    idxs = [prop_types, hut_colors, hut_rotations, tree_types, plant_types,
            windmill_rotations, tower_rotations, tent_rotations, terrain]
    tables = [table_0, table_1, table_2, table_3, table_4, table_5, table_6,
              table_7, table_8]

    iv = [a.reshape(B, WD).astype(jnp.int32) for a in idxs]
    streams = [
        iv[0] * V + iv[1],
        iv[2] * V + iv[3] + 256,
        iv[4] * V + iv[5] + 512,
        iv[6] + 768,
        iv[7] + 768 + V,
        iv[8] + 768 + 2 * V,
    ]
    idx6 = jnp.stack(streams, axis=1)
    idx6 = jnp.pad(idx6, ((0, 0), (0, 0), (0, PWD - WD)))

    tbl = jnp.stack([t.astype(jnp.float32) for t in tables])
    tbl = tbl.at[:8, 0, :].set(0.0)
    pairs = [
        (tbl[2 * k][:, None, :] + tbl[2 * k + 1][None, :, :]).reshape(
            V * V, E)
        for k in range(3)
    ]
    grand = jnp.concatenate(pairs + [tbl[6], tbl[7], tbl[8]])
    gtT = grand.T.reshape(-1)

    out = _embed(gtT, idx6)
    return out.reshape(B, E, W, D)


# speedup vs baseline: 1.0167x; 1.0167x over previous
import functools

import jax
import jax.numpy as jnp
from jax import lax
from jax.experimental import pallas as pl
from jax.experimental.pallas import tpu as pltpu
from jax.experimental.pallas import tpu_sc as plsc

NC = 2
NS = 16
NW = NC * NS
L = 16

B = 1024
W = 25
D = 25
WD = W * D
E = 128
V = 16
NE = 9
NI = 6
GR = 3 * V * V + 3 * V
NG = 40
PWD = NG * L
ECE = 8
NEC = E // ECE


def _body(gt_hbm, idx_hbm, out_hbm, gt_v, idx_v, acc_v, sem0, sem1):
    wid = lax.axis_index("s") * NC + lax.axis_index("c")
    pltpu.sync_copy(gt_hbm, gt_v)
    sems = [sem0, sem1]

    def task_body(t, carry):
        b = t * NW + wid
        pltpu.sync_copy(idx_hbm.at[b], idx_v)

        def chunk2_body(c2, cc):
            for half in range(2):
                ec = c2 * 2 + half
                sem = sems[half]
                dst = out_hbm.at[b, pl.ds(ec * ECE, ECE)]

                @pl.when((c2 > 0) | (t > 0))
                def _wait_prev(half=half, dst=dst, sem=sem):
                    pltpu.make_async_copy(acc_v.at[half], dst, sem).wait()

                @plsc.parallel_loop(0, NG, unroll=5)
                def g_body(g, ec=ec, half=half):
                    o = lax.min(g * L, WD - L)
                    ivs = [idx_v[i, pl.ds(o, L)] for i in range(NI)]
                    for j in range(ECE):
                        base = (ec * ECE + j) * GR
                        win = gt_v.at[pl.ds(base, GR)]
                        gv = [plsc.load_gather(win, [ivs[i]])
                              for i in range(NI)]
                        s = (((gv[0] + gv[1]) + (gv[2] + gv[3]))
                             + (gv[4] + gv[5]))
                        acc_v[half, j, pl.ds(o, L)] = s

                pltpu.async_copy(acc_v.at[half], dst, sem)
            return cc

        lax.fori_loop(0, NEC // 2, chunk2_body, 0)
        return carry

    lax.fori_loop(0, B // NW, task_body, 0)

    for bufi in range(2):
        pltpu.make_async_copy(
            acc_v.at[bufi], out_hbm.at[0, pl.ds(bufi * ECE, ECE)],
            sems[bufi]).wait()


@jax.jit
def _embed(gtT, idx6):
    mesh = plsc.VectorSubcoreMesh(core_axis_name="c", subcore_axis_name="s")
    f = pl.kernel(
        _body,
        out_type=jax.ShapeDtypeStruct((B, E, WD), jnp.float32),
        mesh=mesh,
        scratch_types=[
            pltpu.VMEM((E * GR,), jnp.float32),
            pltpu.VMEM((NI, PWD), jnp.int32),
            pltpu.VMEM((2, ECE, WD), jnp.float32),
            pltpu.SemaphoreType.DMA,
            pltpu.SemaphoreType.DMA,
        ],
        compiler_params=pltpu.CompilerParams(needs_layout_passes=False),
    )
    return f(gtT, idx6)


def kernel(prop_types, hut_colors, hut_rotations, tree_types, plant_types,
           windmill_rotations, tower_rotations, tent_rotations, terrain,
           table_0, table_1, table_2, table_3, table_4, table_5, table_6,
           table_7, table_8):
    idxs = [prop_types, hut_colors, hut_rotations, tree_types, plant_types,
            windmill_rotations, tower_rotations, tent_rotations, terrain]
    tables = [table_0, table_1, table_2, table_3, table_4, table_5, table_6,
              table_7, table_8]

    iv = [a.reshape(B, WD).astype(jnp.int32) for a in idxs]
    streams = [
        iv[0] * V + iv[1],
        iv[2] * V + iv[3] + 256,
        iv[4] * V + iv[5] + 512,
        iv[6] + 768,
        iv[7] + 768 + V,
        iv[8] + 768 + 2 * V,
    ]
    idx6 = jnp.stack(streams, axis=1)
    idx6 = jnp.pad(idx6, ((0, 0), (0, 0), (0, PWD - WD)))

    tbl = jnp.stack([t.astype(jnp.float32) for t in tables])
    tbl = tbl.at[:8, 0, :].set(0.0)
    pairs = [
        (tbl[2 * k][:, None, :] + tbl[2 * k + 1][None, :, :]).reshape(
            V * V, E)
        for k in range(3)
    ]
    grand = jnp.concatenate(pairs + [tbl[6], tbl[7], tbl[8]])
    gtT = grand.T.reshape(-1)

    out = _embed(gtT, idx6)
    return out.reshape(B, E, W, D)


# speedup vs baseline: 1.3465x; 1.3243x over previous
import functools

import jax
import jax.numpy as jnp
from jax import lax
from jax.experimental import pallas as pl
from jax.experimental.pallas import tpu as pltpu
from jax.experimental.pallas import tpu_sc as plsc

NC = 2
NS = 16
NW = NC * NS
L = 16

B = 1024
W = 25
D = 25
WD = W * D
E = 128
V = 16
NE = 9
NI = 6
GR = 3 * V * V + 3 * V
NG = 40
PWD = NG * L
ECE = 8
NEC = E // ECE


def _body(gt_hbm, idx_hbm, out_hbm, gt_v, idx_v, acc_v,
          sem0, sem1, isem0, isem1):
    wid = lax.axis_index("s") * NC + lax.axis_index("c")
    pltpu.sync_copy(gt_hbm, gt_v)
    sems = [sem0, sem1]
    isems = [isem0, isem1]
    NT = B // NW

    pltpu.async_copy(idx_hbm.at[wid], idx_v.at[0], isems[0])

    def row2_body(r2, carry):
        for rh in range(2):
            t = r2 * 2 + rh
            b = t * NW + wid
            idx_b = idx_v.at[rh]

            pltpu.make_async_copy(idx_hbm.at[b], idx_b, isems[rh]).wait()

            @pl.when(t + 1 < NT)
            def _prefetch(t=t, rh=rh):
                pltpu.async_copy(idx_hbm.at[(t + 1) * NW + wid],
                                 idx_v.at[1 - rh], isems[1 - rh])

            def chunk2_body(c2, cc, t=t, b=b, idx_b=idx_b):
                for half in range(2):
                    ec = c2 * 2 + half
                    sem = sems[half]
                    dst = out_hbm.at[b, pl.ds(ec * ECE, ECE)]

                    @pl.when((c2 > 0) | (t > 0))
                    def _wait_prev(half=half, dst=dst, sem=sem):
                        pltpu.make_async_copy(
                            acc_v.at[half], dst, sem).wait()

                    @plsc.parallel_loop(0, NG, unroll=4)
                    def g_body(g, ec=ec, half=half, idx_b=idx_b):
                        o = lax.min(g * L, WD - L)
                        ivs = [idx_b[i, pl.ds(o, L)] for i in range(NI)]
                        for j in range(ECE):
                            base = (ec * ECE + j) * GR
                            win = gt_v.at[pl.ds(base, GR)]
                            gv = [plsc.load_gather(win, [ivs[i]])
                                  for i in range(NI)]
                            s = (((gv[0] + gv[1]) + (gv[2] + gv[3]))
                                 + (gv[4] + gv[5]))
                            acc_v[half, j, pl.ds(o, L)] = s

                    pltpu.async_copy(acc_v.at[half], dst, sem)
                return cc

            lax.fori_loop(0, NEC // 2, chunk2_body, 0)
        return carry

    lax.fori_loop(0, NT // 2, row2_body, 0)

    for bufi in range(2):
        pltpu.make_async_copy(
            acc_v.at[bufi], out_hbm.at[0, pl.ds(bufi * ECE, ECE)],
            sems[bufi]).wait()


@jax.jit
def _embed(gtT, idx6):
    mesh = plsc.VectorSubcoreMesh(core_axis_name="c", subcore_axis_name="s")
    f = pl.kernel(
        _body,
        out_type=jax.ShapeDtypeStruct((B, E, WD), jnp.float32),
        mesh=mesh,
        scratch_types=[
            pltpu.VMEM((E * GR,), jnp.float32),
            pltpu.VMEM((2, NI, PWD), jnp.int32),
            pltpu.VMEM((2, ECE, WD), jnp.float32),
            pltpu.SemaphoreType.DMA,
            pltpu.SemaphoreType.DMA,
            pltpu.SemaphoreType.DMA,
            pltpu.SemaphoreType.DMA,
        ],
        compiler_params=pltpu.CompilerParams(needs_layout_passes=False),
    )
    return f(gtT, idx6)


def kernel(prop_types, hut_colors, hut_rotations, tree_types, plant_types,
           windmill_rotations, tower_rotations, tent_rotations, terrain,
           table_0, table_1, table_2, table_3, table_4, table_5, table_6,
           table_7, table_8):
    idxs = [prop_types, hut_colors, hut_rotations, tree_types, plant_types,
            windmill_rotations, tower_rotations, tent_rotations, terrain]
    tables = [table_0, table_1, table_2, table_3, table_4, table_5, table_6,
              table_7, table_8]

    iv = [a.reshape(B, WD).astype(jnp.int32) for a in idxs]
    streams = [
        iv[0] * V + iv[1],
        iv[2] * V + iv[3] + 256,
        iv[4] * V + iv[5] + 512,
        iv[6] + 768,
        iv[7] + 768 + V,
        iv[8] + 768 + 2 * V,
    ]
    idx6 = jnp.stack(streams, axis=1)
    idx6 = jnp.pad(idx6, ((0, 0), (0, 0), (0, PWD - WD)))

    tbl = jnp.stack([t.astype(jnp.float32) for t in tables])
    tbl = tbl.at[:8, 0, :].set(0.0)
    pairs = [
        (tbl[2 * k][:, None, :] + tbl[2 * k + 1][None, :, :]).reshape(
            V * V, E)
        for k in range(3)
    ]
    grand = jnp.concatenate(pairs + [tbl[6], tbl[7], tbl[8]])
    gtT = grand.T.reshape(-1)

    out = _embed(gtT, idx6)
    return out.reshape(B, E, W, D)
